# baseline (speedup 1.0000x reference)
import functools

import jax
import jax.numpy as jnp
from jax import lax
from jax.experimental import pallas as pl
from jax.experimental.pallas import tpu as pltpu
from jax.experimental.pallas import tpu_sc as plsc

N = 10000
E = 320000
H = 128
AF = 128
BF = 16
G = 256
DEPTH = 3

NC = 2
NS = 16
NW = NC * NS

NP = 10240
BR = 2048
EC = 80
ECH = E // NW // EC
EW = E // NW
NCEIL = NP
MRW = NCEIL // NS
B = E // 2
PC = 80
PCH = NP // NW // PC
PR = NP // NW
GR = G // NS

_MESH = plsc.VectorSubcoreMesh(
    core_axis_name="c", subcore_axis_name="s", num_cores=NC, num_subcores=NS)


def _row_mask(val):
    rows = lax.broadcasted_iota(jnp.int32, val.shape, 0) + pl.program_id(0) * BR
    return jnp.where(rows < N, val, 0.0)


def _in_body(x_ref, w_ref, b_ref, o_ref):
    v = jnp.dot(x_ref[...], w_ref[...], preferred_element_type=jnp.float32)
    o_ref[...] = _row_mask(jnp.maximum(v + b_ref[...], 0.0))


_tc_in = pl.pallas_call(
    _in_body,
    grid=(NP // BR,),
    in_specs=[
        pl.BlockSpec((BR, AF), lambda i: (i, 0)),
        pl.BlockSpec((AF, H), lambda i: (0, 0)),
        pl.BlockSpec((1, H), lambda i: (0, 0)),
    ],
    out_specs=pl.BlockSpec((BR, H), lambda i: (i, 0)),
    out_shape=jax.ShapeDtypeStruct((NP, H), jnp.float32),
)


def _round_body(h_ref, m0_ref, m1_ref, w_ref, b_ref, o_ref):
    m = m0_ref[0] + m1_ref[0]
    v = jnp.dot(m, w_ref[...], preferred_element_type=jnp.float32)
    o_ref[...] = _row_mask(jnp.maximum(h_ref[...] + v + b_ref[...], 0.0))


_tc_round = pl.pallas_call(
    _round_body,
    grid=(NP // BR,),
    in_specs=[
        pl.BlockSpec((BR, H), lambda i: (i, 0)),
        pl.BlockSpec((1, BR, H), lambda i: (0, i, 0)),
        pl.BlockSpec((1, BR, H), lambda i: (1, i, 0)),
        pl.BlockSpec((H, H), lambda i: (0, 0)),
        pl.BlockSpec((1, H), lambda i: (0, 0)),
    ],
    out_specs=pl.BlockSpec((BR, H), lambda i: (i, 0)),
    out_shape=jax.ShapeDtypeStruct((NP, H), jnp.float32),
)


def _heads_body(h_ref, wn_ref, bn_ref, we_ref, be_ref, np_ref, hp_ref):
    hh = h_ref[...]
    np_ref[...] = jnp.dot(hh, wn_ref[...], preferred_element_type=jnp.float32) + bn_ref[...]
    ep = jnp.dot(hh, we_ref[...], preferred_element_type=jnp.float32)
    hp_ref[...] = 0.5 * (ep + be_ref[...])


_tc_heads = pl.pallas_call(
    _heads_body,
    grid=(NP // BR,),
    in_specs=[
        pl.BlockSpec((BR, H), lambda i: (i, 0)),
        pl.BlockSpec((H, AF), lambda i: (0, 0)),
        pl.BlockSpec((1, AF), lambda i: (0, 0)),
        pl.BlockSpec((H, BF), lambda i: (0, 0)),
        pl.BlockSpec((1, BF), lambda i: (0, 0)),
    ],
    out_specs=[
        pl.BlockSpec((BR, AF), lambda i: (i, 0)),
        pl.BlockSpec((BR, BF), lambda i: (i, 0)),
    ],
    out_shape=[
        jax.ShapeDtypeStruct((NP, AF), jnp.float32),
        jax.ShapeDtypeStruct((NP, BF), jnp.float32),
    ],
)


def _graph_body(g0_ref, g1_ref, w1_ref, b1_ref, w2_ref, b2_ref, o_ref):
    gsum = g0_ref[0] + g1_ref[0]
    t = jnp.maximum(
        jnp.dot(gsum, w1_ref[...], preferred_element_type=jnp.float32)
        + b1_ref[...], 0.0)
    o_ref[...] = jnp.dot(t, w2_ref[...], preferred_element_type=jnp.float32) + b2_ref[...]


_tc_graph = pl.pallas_call(
    _graph_body,
    grid=(1,),
    in_specs=[
        pl.BlockSpec((1, G, H), lambda i: (0, 0, 0)),
        pl.BlockSpec((1, G, H), lambda i: (1, 0, 0)),
        pl.BlockSpec((H, H), lambda i: (0, 0)),
        pl.BlockSpec((1, H), lambda i: (0, 0)),
        pl.BlockSpec((H, H), lambda i: (0, 0)),
        pl.BlockSpec((1, H), lambda i: (0, 0)),
    ],
    out_specs=pl.BlockSpec((G, H), lambda i: (0, 0)),
    out_shape=jax.ShapeDtypeStruct((G, H), jnp.float32),
)


@functools.partial(
    pl.kernel,
    out_type=jax.ShapeDtypeStruct((NC, NP, H), jnp.float32),
    mesh=_MESH,
    scratch_types=[
        pltpu.VMEM_SHARED((NCEIL, H), jnp.float32),
        pltpu.VMEM((EW,), jnp.int32),
        pltpu.VMEM((ECH, EC), jnp.int32),
        pltpu.VMEM((2, EC, H), jnp.float32),
        pltpu.SemaphoreType.DMA,
        pltpu.SemaphoreType.DMA,
        pltpu.SemaphoreType.DMA,
        pltpu.SemaphoreType.DMA,
    ],
)
def _sc_msg(h_hbm, src_hbm, dst_hbm, out_hbm, m_sh, src_v, dst_v, rows_v,
            gsem0, gsem1, ssem0, ssem1):
    c = lax.axis_index("c")
    s = lax.axis_index("s")
    w = c * NS + s
    gsem = (gsem0, gsem1)
    ssem = (ssem0, ssem1)

    zero16 = jnp.zeros((16,), jnp.float32)

    def _zrow(i, carry):
        for j in range(H // 16):
            rows_v[0, i, pl.ds(j * 16, 16)] = zero16
        return carry

    lax.fori_loop(0, EC, _zrow, 0)
    for k in range(MRW // EC):
        pltpu.sync_copy(rows_v.at[0], m_sh.at[pl.ds(s * MRW + k * EC, EC)])
    pltpu.sync_copy(src_hbm.at[w], src_v)
    pltpu.sync_copy(dst_hbm.at[w], dst_v)
    plsc.subcore_barrier()

    def _gather(j, b, sem):
        return pltpu.make_async_copy(
            h_hbm.at[src_v.at[pl.ds(j * EC, EC)]], rows_v.at[b], sem)

    def _gstart(j, b, sem):
        pltpu.async_copy(
            h_hbm.at[src_v.at[pl.ds(j * EC, EC)]], rows_v.at[b], sem)

    def _scat(j, b):
        return pltpu.make_async_copy(
            rows_v.at[b], m_sh.at[dst_v.at[j]], ssem[b])

    _gstart(0, 0, gsem0)

    def _step(jj, carry):
        for b in (0, 1):
            j = jj * 2 + b
            nb = 1 - b
            _gather(j, b, gsem[b]).wait()

            @pl.when(j > 0)
            def _():
                _scat(j - 1, nb).wait()

            _gstart(j + 1, nb, gsem[nb])
            pltpu.async_copy(rows_v.at[b], m_sh.at[dst_v.at[j]], ssem[b],
                             add=True)
        return carry

    lax.fori_loop(0, (ECH - 1) // 2, _step, 0)
    _gather(ECH - 1, 0, gsem0).wait()
    _scat(ECH - 2, 1).wait()
    pltpu.async_copy(rows_v.at[0], m_sh.at[dst_v.at[ECH - 1]], ssem0, add=True)
    _scat(ECH - 1, 0).wait()
    plsc.subcore_barrier()
    pltpu.sync_copy(m_sh.at[pl.ds(s * MRW, MRW)],
                    out_hbm.at[c, pl.ds(s * MRW, MRW)])


@functools.partial(
    pl.kernel,
    out_type=(
        jax.ShapeDtypeStruct((NC, G, H), jnp.float32),
        jax.ShapeDtypeStruct((B, BF), jnp.float32),
    ),
    mesh=_MESH,
    scratch_types=[
        pltpu.VMEM_SHARED((G, H), jnp.float32),
        pltpu.VMEM((PCH, PC), jnp.int32),
        pltpu.VMEM((PC, H), jnp.float32),
        pltpu.VMEM((EW,), jnp.int32),
        pltpu.VMEM((2, EC, BF), jnp.float32),
        pltpu.VMEM((EC // 2, BF), jnp.float32),
        pltpu.SemaphoreType.DMA,
        pltpu.SemaphoreType.DMA,
    ],
    compiler_params=pltpu.CompilerParams(use_tc_tiling_on_sc=False),
)
def _sc_heads(h_hbm, hp_hbm, batch_hbm, src_hbm, g_hbm, ep_hbm,
              g_sh, bidx_v, prow_v, sidx_v, ebuf_v, eo_v, esem0, esem1):
    c = lax.axis_index("c")
    s = lax.axis_index("s")
    w = c * NS + s
    esem = (esem0, esem1)
    zero16 = jnp.zeros((16,), jnp.float32)

    def _zrow(i, carry):
        for j in range(H // 16):
            prow_v[i, pl.ds(j * 16, 16)] = zero16
        return carry

    lax.fori_loop(0, GR, _zrow, 0)
    pltpu.sync_copy(prow_v.at[pl.ds(0, GR)], g_sh.at[pl.ds(s * GR, GR)])
    pltpu.sync_copy(batch_hbm.at[w], bidx_v)
    pltpu.sync_copy(src_hbm.at[w], sidx_v)
    plsc.subcore_barrier()

    def _pstep(k, carry):
        pltpu.sync_copy(h_hbm.at[pl.ds(w * PR + k * PC, PC)], prow_v)
        pltpu.sync_copy(prow_v, g_sh.at[bidx_v.at[k]], add=True)
        return carry

    lax.fori_loop(0, PCH, _pstep, 0)

    def _egather(j, b, sem):
        return pltpu.make_async_copy(
            hp_hbm.at[sidx_v.at[pl.ds(j * EC, EC)]], ebuf_v.at[b], sem)

    def _esum_out(j, b):
        for r in range(EC // 2):
            eo_v[r] = ebuf_v[b, 2 * r] + ebuf_v[b, 2 * r + 1]
        pltpu.sync_copy(
            eo_v, ep_hbm.at[pl.ds(w * (B // NW) + j * (EC // 2), EC // 2)])

    pltpu.async_copy(
        hp_hbm.at[sidx_v.at[pl.ds(0, EC)]], ebuf_v.at[0], esem0)

    def _estep(jj, carry):
        for b in (0, 1):
            j = jj * 2 + b
            _egather(j, b, esem[b]).wait()
            pltpu.async_copy(
                hp_hbm.at[sidx_v.at[pl.ds((j + 1) * EC, EC)]],
                ebuf_v.at[1 - b], esem[1 - b])
            _esum_out(j, b)
        return carry

    lax.fori_loop(0, (ECH - 1) // 2, _estep, 0)
    _egather(ECH - 1, 0, esem0).wait()
    _esum_out(ECH - 1, 0)

    plsc.subcore_barrier()
    pltpu.sync_copy(g_sh.at[pl.ds(s * GR, GR)], g_hbm.at[c, pl.ds(s * GR, GR)])


def kernel(x, edge_index, rev_edge_index, batch,
           W_in, b_in, W_h, b_h, W_node, b_node,
           W_edge, b_edge, W_g1, b_g1, W_g2, b_g2):
    del rev_edge_index
    src2 = edge_index[0].astype(jnp.int32).reshape(NW, EW)
    dst3 = edge_index[1].astype(jnp.int32).reshape(NW, ECH, EC)
    batch3 = jnp.concatenate(
        [batch.astype(jnp.int32), jnp.zeros((NP - N,), jnp.int32)]
    ).reshape(NW, PCH, PC)
    xp = jnp.pad(x.astype(jnp.float32), ((0, NP - N), (0, 0)))

    h = _tc_in(xp, W_in, b_in.reshape(1, H))
    for _ in range(DEPTH):
        parts = _sc_msg(h, src2, dst3)
        h = _tc_round(h, parts, parts, W_h, b_h.reshape(1, H))

    node_pred, hp = _tc_heads(
        h, W_node, b_node.reshape(1, AF), W_edge, b_edge.reshape(1, BF))
    g, edge_pred = _sc_heads(h, hp, batch3, src2)
    gfull = _tc_graph(
        g, g, W_g1, b_g1.reshape(1, H),
        jnp.pad(W_g2, ((0, 0), (0, H - 1))),
        jnp.pad(b_g2, (0, H - 1)).reshape(1, H))
    return node_pred[:N], edge_pred, gfull[:, :1]


# speedup vs baseline: 1.3191x; 1.3191x over previous
import functools

import jax
import jax.numpy as jnp
from jax import lax
from jax.experimental import pallas as pl
from jax.experimental.pallas import tpu as pltpu
from jax.experimental.pallas import tpu_sc as plsc

N = 10000
E = 320000
H = 128
AF = 128
BF = 16
G = 256
DEPTH = 3

NC = 2
NS = 16
NW = NC * NS

NP = 10240
BR = 2048
EC = 80
ECH = E // NW // EC
EW = E // NW
NCEIL = N
MRW = NCEIL // NS
B = E // 2
PC = 80
PCH = NP // NW // PC
PR = NP // NW
GR = G // NS

_MESH = plsc.VectorSubcoreMesh(
    core_axis_name="c", subcore_axis_name="s", num_cores=NC, num_subcores=NS)


def _row_mask(val):
    rows = lax.broadcasted_iota(jnp.int32, val.shape, 0) + pl.program_id(0) * BR
    return jnp.where(rows < N, val, 0.0)


def _in_body(x_ref, w_ref, b_ref, o_ref):
    v = jnp.dot(x_ref[...], w_ref[...], preferred_element_type=jnp.float32)
    o_ref[...] = _row_mask(jnp.maximum(v + b_ref[...], 0.0))


_tc_in = pl.pallas_call(
    _in_body,
    grid=(NP // BR,),
    in_specs=[
        pl.BlockSpec((BR, AF), lambda i: (i, 0)),
        pl.BlockSpec((AF, H), lambda i: (0, 0)),
        pl.BlockSpec((1, H), lambda i: (0, 0)),
    ],
    out_specs=pl.BlockSpec((BR, H), lambda i: (i, 0)),
    out_shape=jax.ShapeDtypeStruct((NP, H), jnp.float32),
)


def _round_body(h_ref, m0_ref, m1_ref, w_ref, b_ref, o_ref):
    m = m0_ref[0] + m1_ref[0]
    v = jnp.dot(m, w_ref[...], preferred_element_type=jnp.float32)
    o_ref[...] = _row_mask(jnp.maximum(h_ref[...] + v + b_ref[...], 0.0))


_tc_round = pl.pallas_call(
    _round_body,
    grid=(NP // BR,),
    in_specs=[
        pl.BlockSpec((BR, H), lambda i: (i, 0)),
        pl.BlockSpec((1, BR, H), lambda i: (0, i, 0)),
        pl.BlockSpec((1, BR, H), lambda i: (1, i, 0)),
        pl.BlockSpec((H, H), lambda i: (0, 0)),
        pl.BlockSpec((1, H), lambda i: (0, 0)),
    ],
    out_specs=pl.BlockSpec((BR, H), lambda i: (i, 0)),
    out_shape=jax.ShapeDtypeStruct((NP, H), jnp.float32),
)


def _heads_body(h_ref, wn_ref, bn_ref, we_ref, be_ref, np_ref, hp_ref):
    hh = h_ref[...]
    np_ref[...] = jnp.dot(hh, wn_ref[...], preferred_element_type=jnp.float32) + bn_ref[...]
    ep = jnp.dot(hh, we_ref[...], preferred_element_type=jnp.float32)
    hp_ref[...] = 0.5 * (ep + be_ref[...])


_tc_heads = pl.pallas_call(
    _heads_body,
    grid=(NP // BR,),
    in_specs=[
        pl.BlockSpec((BR, H), lambda i: (i, 0)),
        pl.BlockSpec((H, AF), lambda i: (0, 0)),
        pl.BlockSpec((1, AF), lambda i: (0, 0)),
        pl.BlockSpec((H, BF), lambda i: (0, 0)),
        pl.BlockSpec((1, BF), lambda i: (0, 0)),
    ],
    out_specs=[
        pl.BlockSpec((BR, AF), lambda i: (i, 0)),
        pl.BlockSpec((BR, BF), lambda i: (i, 0)),
    ],
    out_shape=[
        jax.ShapeDtypeStruct((NP, AF), jnp.float32),
        jax.ShapeDtypeStruct((NP, BF), jnp.float32),
    ],
)


def _graph_body(g0_ref, g1_ref, w1_ref, b1_ref, w2_ref, b2_ref, o_ref):
    gsum = g0_ref[0] + g1_ref[0]
    t = jnp.maximum(
        jnp.dot(gsum, w1_ref[...], preferred_element_type=jnp.float32)
        + b1_ref[...], 0.0)
    o_ref[...] = jnp.dot(t, w2_ref[...], preferred_element_type=jnp.float32) + b2_ref[...]


_tc_graph = pl.pallas_call(
    _graph_body,
    grid=(1,),
    in_specs=[
        pl.BlockSpec((1, G, H), lambda i: (0, 0, 0)),
        pl.BlockSpec((1, G, H), lambda i: (1, 0, 0)),
        pl.BlockSpec((H, H), lambda i: (0, 0)),
        pl.BlockSpec((1, H), lambda i: (0, 0)),
        pl.BlockSpec((H, H), lambda i: (0, 0)),
        pl.BlockSpec((1, H), lambda i: (0, 0)),
    ],
    out_specs=pl.BlockSpec((G, H), lambda i: (0, 0)),
    out_shape=jax.ShapeDtypeStruct((G, H), jnp.float32),
)


@functools.partial(
    pl.kernel,
    out_type=jax.ShapeDtypeStruct((NC, NP, H), jnp.float32),
    mesh=_MESH,
    scratch_types=[
        pltpu.VMEM_SHARED((NCEIL, H), jnp.float32),
        pltpu.VMEM((EW,), jnp.int32),
        pltpu.VMEM((EW,), jnp.int32),
        pltpu.VMEM((3, EC, H), jnp.float32),
        pltpu.SemaphoreType.DMA,
        pltpu.SemaphoreType.DMA,
        pltpu.SemaphoreType.DMA,
        pltpu.SemaphoreType.DMA,
        pltpu.SemaphoreType.DMA,
        pltpu.SemaphoreType.DMA,
    ],
    compiler_params=pltpu.CompilerParams(use_tc_tiling_on_sc=False),
)
def _sc_msg(h_hbm, src_hbm, dst_hbm, out_hbm, m_sh, src_v, dst_v, rows_v,
            gsem0, gsem1, gsem2, ssem0, ssem1, ssem2):
    c = lax.axis_index("c")
    s = lax.axis_index("s")
    w = c * NS + s
    gsem = (gsem0, gsem1, gsem2)
    ssem = (ssem0, ssem1, ssem2)

    zero16 = jnp.zeros((16,), jnp.float32)

    def _zrow(i, carry):
        for j in range(H // 16):
            rows_v[0, i, pl.ds(j * 16, 16)] = zero16
        return carry

    lax.fori_loop(0, EC, _zrow, 0)
    for k in range(MRW // EC):
        pltpu.sync_copy(rows_v.at[0], m_sh.at[pl.ds(s * MRW + k * EC, EC)])
    rem = MRW % EC
    if rem:
        pltpu.sync_copy(rows_v.at[0, pl.ds(0, rem), :],
                        m_sh.at[pl.ds(s * MRW + (MRW // EC) * EC, rem)])
    pltpu.sync_copy(src_hbm.at[w], src_v)
    pltpu.sync_copy(dst_hbm.at[w], dst_v)
    plsc.subcore_barrier()

    def _gather(j, b):
        return pltpu.make_async_copy(
            h_hbm.at[src_v.at[pl.ds(j * EC, EC)]], rows_v.at[b], gsem[b])

    def _gstart(j, b):
        pltpu.async_copy(
            h_hbm.at[src_v.at[pl.ds(j * EC, EC)]], rows_v.at[b], gsem[b])

    def _scat(j, b):
        return pltpu.make_async_copy(
            rows_v.at[b], m_sh.at[dst_v.at[pl.ds(j * EC, EC)]], ssem[b])

    _gstart(0, 0)
    _gstart(1, 1)

    def _step(jj, carry):
        for b in (0, 1, 2):
            j = jj * 3 + b
            _gather(j, b).wait()

            @pl.when(j >= 1)
            def _():
                _scat(j - 1, (b - 1) % 3).wait()

            _gstart(j + 2, (b + 2) % 3)
            pltpu.async_copy(rows_v.at[b],
                             m_sh.at[dst_v.at[pl.ds(j * EC, EC)]], ssem[b],
                             add=True)
        return carry

    lax.fori_loop(0, (ECH - 2) // 3, _step, 0)
    for j in (ECH - 2, ECH - 1):
        b = j % 3
        _gather(j, b).wait()
        _scat(j - 1, (j - 1) % 3).wait()
        pltpu.async_copy(rows_v.at[b],
                         m_sh.at[dst_v.at[pl.ds(j * EC, EC)]], ssem[b],
                         add=True)
    _scat(ECH - 1, (ECH - 1) % 3).wait()
    plsc.subcore_barrier()
    pltpu.sync_copy(m_sh.at[pl.ds(s * MRW, MRW)],
                    out_hbm.at[c, pl.ds(s * MRW, MRW)])


@functools.partial(
    pl.kernel,
    out_type=(
        jax.ShapeDtypeStruct((NC, G, H), jnp.float32),
        jax.ShapeDtypeStruct((B, BF), jnp.float32),
    ),
    mesh=_MESH,
    scratch_types=[
        pltpu.VMEM_SHARED((G, H), jnp.float32),
        pltpu.VMEM((PCH, PC), jnp.int32),
        pltpu.VMEM((PC, H), jnp.float32),
        pltpu.VMEM((EW,), jnp.int32),
        pltpu.VMEM((2, EC, BF), jnp.float32),
        pltpu.VMEM((EC // 2, BF), jnp.float32),
        pltpu.SemaphoreType.DMA,
        pltpu.SemaphoreType.DMA,
    ],
    compiler_params=pltpu.CompilerParams(use_tc_tiling_on_sc=False),
)
def _sc_heads(h_hbm, hp_hbm, batch_hbm, src_hbm, g_hbm, ep_hbm,
              g_sh, bidx_v, prow_v, sidx_v, ebuf_v, eo_v, esem0, esem1):
    c = lax.axis_index("c")
    s = lax.axis_index("s")
    w = c * NS + s
    esem = (esem0, esem1)
    zero16 = jnp.zeros((16,), jnp.float32)

    def _zrow(i, carry):
        for j in range(H // 16):
            prow_v[i, pl.ds(j * 16, 16)] = zero16
        return carry

    lax.fori_loop(0, GR, _zrow, 0)
    pltpu.sync_copy(prow_v.at[pl.ds(0, GR)], g_sh.at[pl.ds(s * GR, GR)])
    pltpu.sync_copy(batch_hbm.at[w], bidx_v)
    pltpu.sync_copy(src_hbm.at[w], sidx_v)
    plsc.subcore_barrier()

    def _pstep(k, carry):
        pltpu.sync_copy(h_hbm.at[pl.ds(w * PR + k * PC, PC)], prow_v)
        pltpu.sync_copy(prow_v, g_sh.at[bidx_v.at[k]], add=True)
        return carry

    lax.fori_loop(0, PCH, _pstep, 0)

    def _egather(j, b, sem):
        return pltpu.make_async_copy(
            hp_hbm.at[sidx_v.at[pl.ds(j * EC, EC)]], ebuf_v.at[b], sem)

    def _esum_out(j, b):
        for r in range(EC // 2):
            eo_v[r] = ebuf_v[b, 2 * r] + ebuf_v[b, 2 * r + 1]
        pltpu.sync_copy(
            eo_v, ep_hbm.at[pl.ds(w * (B // NW) + j * (EC // 2), EC // 2)])

    pltpu.async_copy(
        hp_hbm.at[sidx_v.at[pl.ds(0, EC)]], ebuf_v.at[0], esem0)

    def _estep(jj, carry):
        for b in (0, 1):
            j = jj * 2 + b
            _egather(j, b, esem[b]).wait()
            pltpu.async_copy(
                hp_hbm.at[sidx_v.at[pl.ds((j + 1) * EC, EC)]],
                ebuf_v.at[1 - b], esem[1 - b])
            _esum_out(j, b)
        return carry

    lax.fori_loop(0, (ECH - 1) // 2, _estep, 0)
    _egather(ECH - 1, 0, esem0).wait()
    _esum_out(ECH - 1, 0)

    plsc.subcore_barrier()
    pltpu.sync_copy(g_sh.at[pl.ds(s * GR, GR)], g_hbm.at[c, pl.ds(s * GR, GR)])


def kernel(x, edge_index, rev_edge_index, batch,
           W_in, b_in, W_h, b_h, W_node, b_node,
           W_edge, b_edge, W_g1, b_g1, W_g2, b_g2):
    del rev_edge_index
    src2 = edge_index[0].astype(jnp.int32).reshape(NW, EW)
    dst2 = edge_index[1].astype(jnp.int32).reshape(NW, EW)
    batch3 = jnp.concatenate(
        [batch.astype(jnp.int32), jnp.zeros((NP - N,), jnp.int32)]
    ).reshape(NW, PCH, PC)
    xp = jnp.pad(x.astype(jnp.float32), ((0, NP - N), (0, 0)))

    h = _tc_in(xp, W_in, b_in.reshape(1, H))
    for _ in range(DEPTH):
        parts = _sc_msg(h, src2, dst2)
        h = _tc_round(h, parts, parts, W_h, b_h.reshape(1, H))

    node_pred, hp = _tc_heads(
        h, W_node, b_node.reshape(1, AF), W_edge, b_edge.reshape(1, BF))
    g, edge_pred = _sc_heads(h, hp, batch3, src2)
    gfull = _tc_graph(
        g, g, W_g1, b_g1.reshape(1, H),
        jnp.pad(W_g2, ((0, 0), (0, H - 1))),
        jnp.pad(b_g2, (0, H - 1)).reshape(1, H))
    return node_pred[:N], edge_pred, gfull[:, :1]


# speedup vs baseline: 1.4146x; 1.0723x over previous
import functools

import jax
import jax.numpy as jnp
from jax import lax
from jax.experimental import pallas as pl
from jax.experimental.pallas import tpu as pltpu
from jax.experimental.pallas import tpu_sc as plsc

N = 10000
E = 320000
H = 128
AF = 128
BF = 16
G = 256
DEPTH = 3

NC = 2
NS = 16
NW = NC * NS

NP = 10240
BR = 2048
EC = 80
ECH = E // NW // EC
EW = E // NW
NCEIL = N
MRW = NCEIL // NS
B = E // 2
PC = 80
PCH = NP // NW // PC
PR = NP // NW
GR = G // NS

_MESH = plsc.VectorSubcoreMesh(
    core_axis_name="c", subcore_axis_name="s", num_cores=NC, num_subcores=NS)


def _row_mask(val):
    rows = lax.broadcasted_iota(jnp.int32, val.shape, 0) + pl.program_id(0) * BR
    return jnp.where(rows < N, val, 0.0)


def _in_body(x_ref, w_ref, b_ref, o_ref):
    v = jnp.dot(x_ref[...], w_ref[...], preferred_element_type=jnp.float32)
    o_ref[...] = _row_mask(jnp.maximum(v + b_ref[...], 0.0))


_tc_in = pl.pallas_call(
    _in_body,
    grid=(NP // BR,),
    in_specs=[
        pl.BlockSpec((BR, AF), lambda i: (i, 0)),
        pl.BlockSpec((AF, H), lambda i: (0, 0)),
        pl.BlockSpec((1, H), lambda i: (0, 0)),
    ],
    out_specs=pl.BlockSpec((BR, H), lambda i: (i, 0)),
    out_shape=jax.ShapeDtypeStruct((NP, H), jnp.float32),
)


def _round_body(h_ref, m0_ref, m1_ref, w_ref, b_ref, o_ref):
    m = m0_ref[0] + m1_ref[0]
    v = jnp.dot(m, w_ref[...], preferred_element_type=jnp.float32)
    o_ref[...] = _row_mask(jnp.maximum(h_ref[...] + v + b_ref[...], 0.0))


_tc_round = pl.pallas_call(
    _round_body,
    grid=(NP // BR,),
    in_specs=[
        pl.BlockSpec((BR, H), lambda i: (i, 0)),
        pl.BlockSpec((1, BR, H), lambda i: (0, i, 0)),
        pl.BlockSpec((1, BR, H), lambda i: (1, i, 0)),
        pl.BlockSpec((H, H), lambda i: (0, 0)),
        pl.BlockSpec((1, H), lambda i: (0, 0)),
    ],
    out_specs=pl.BlockSpec((BR, H), lambda i: (i, 0)),
    out_shape=jax.ShapeDtypeStruct((NP, H), jnp.float32),
)


def _heads_body(h_ref, wn_ref, bn_ref, we_ref, be_ref, np_ref, hp_ref):
    hh = h_ref[...]
    np_ref[...] = jnp.dot(hh, wn_ref[...], preferred_element_type=jnp.float32) + bn_ref[...]
    ep = jnp.dot(hh, we_ref[...], preferred_element_type=jnp.float32)
    hp_ref[...] = 0.5 * (ep + be_ref[...])


_tc_heads = pl.pallas_call(
    _heads_body,
    grid=(NP // BR,),
    in_specs=[
        pl.BlockSpec((BR, H), lambda i: (i, 0)),
        pl.BlockSpec((H, AF), lambda i: (0, 0)),
        pl.BlockSpec((1, AF), lambda i: (0, 0)),
        pl.BlockSpec((H, BF), lambda i: (0, 0)),
        pl.BlockSpec((1, BF), lambda i: (0, 0)),
    ],
    out_specs=[
        pl.BlockSpec((BR, AF), lambda i: (i, 0)),
        pl.BlockSpec((BR, BF), lambda i: (i, 0)),
    ],
    out_shape=[
        jax.ShapeDtypeStruct((NP, AF), jnp.float32),
        jax.ShapeDtypeStruct((NP, BF), jnp.float32),
    ],
)


def _graph_body(g0_ref, g1_ref, w1_ref, b1_ref, w2_ref, b2_ref, o_ref):
    gsum = g0_ref[0] + g1_ref[0]
    t = jnp.maximum(
        jnp.dot(gsum, w1_ref[...], preferred_element_type=jnp.float32)
        + b1_ref[...], 0.0)
    o_ref[...] = jnp.dot(t, w2_ref[...], preferred_element_type=jnp.float32) + b2_ref[...]


_tc_graph = pl.pallas_call(
    _graph_body,
    grid=(1,),
    in_specs=[
        pl.BlockSpec((1, G, H), lambda i: (0, 0, 0)),
        pl.BlockSpec((1, G, H), lambda i: (1, 0, 0)),
        pl.BlockSpec((H, H), lambda i: (0, 0)),
        pl.BlockSpec((1, H), lambda i: (0, 0)),
        pl.BlockSpec((H, H), lambda i: (0, 0)),
        pl.BlockSpec((1, H), lambda i: (0, 0)),
    ],
    out_specs=pl.BlockSpec((G, H), lambda i: (0, 0)),
    out_shape=jax.ShapeDtypeStruct((G, H), jnp.float32),
)


@functools.partial(
    pl.kernel,
    out_type=jax.ShapeDtypeStruct((NC, NP, H), jnp.float32),
    mesh=_MESH,
    scratch_types=[
        pltpu.VMEM_SHARED((NCEIL, H), jnp.float32),
        pltpu.VMEM((EW,), jnp.int32),
        pltpu.VMEM((EW,), jnp.int32),
        pltpu.VMEM((3, EC, H), jnp.float32),
        pltpu.SemaphoreType.DMA,
        pltpu.SemaphoreType.DMA,
        pltpu.SemaphoreType.DMA,
        pltpu.SemaphoreType.DMA,
        pltpu.SemaphoreType.DMA,
        pltpu.SemaphoreType.DMA,
    ],
    compiler_params=pltpu.CompilerParams(use_tc_tiling_on_sc=False),
)
def _sc_msg(h_hbm, src_hbm, dst_hbm, out_hbm, m_sh, src_v, dst_v, rows_v,
            gsem0, gsem1, gsem2, ssem0, ssem1, ssem2):
    c = lax.axis_index("c")
    s = lax.axis_index("s")
    w = c * NS + s
    gsem = (gsem0, gsem1, gsem2)
    ssem = (ssem0, ssem1, ssem2)

    zero16 = jnp.zeros((16,), jnp.float32)

    def _zrow(i, carry):
        for j in range(H // 16):
            rows_v[0, i, pl.ds(j * 16, 16)] = zero16
        return carry

    lax.fori_loop(0, EC, _zrow, 0)
    for k in range(MRW // EC):
        pltpu.sync_copy(rows_v.at[0], m_sh.at[pl.ds(s * MRW + k * EC, EC)])
    rem = MRW % EC
    if rem:
        pltpu.sync_copy(rows_v.at[0, pl.ds(0, rem), :],
                        m_sh.at[pl.ds(s * MRW + (MRW // EC) * EC, rem)])
    pltpu.sync_copy(src_hbm.at[w], src_v)
    pltpu.sync_copy(dst_hbm.at[w], dst_v)
    plsc.subcore_barrier()

    def _gather(j, b):
        return pltpu.make_async_copy(
            h_hbm.at[src_v.at[pl.ds(j * EC, EC)]], rows_v.at[b], gsem[b])

    def _gstart(j, b):
        pltpu.async_copy(
            h_hbm.at[src_v.at[pl.ds(j * EC, EC)]], rows_v.at[b], gsem[b])

    def _scat(j, b):
        return pltpu.make_async_copy(
            rows_v.at[b], m_sh.at[dst_v.at[pl.ds(j * EC, EC)]], ssem[b])

    _gstart(0, 0)
    _gstart(1, 1)

    def _step(jj, carry):
        for b in (0, 1, 2):
            j = jj * 3 + b
            _gather(j, b).wait()

            @pl.when(j >= 1)
            def _():
                _scat(j - 1, (b - 1) % 3).wait()

            _gstart(j + 2, (b + 2) % 3)
            pltpu.async_copy(rows_v.at[b],
                             m_sh.at[dst_v.at[pl.ds(j * EC, EC)]], ssem[b],
                             add=True)
        return carry

    lax.fori_loop(0, (ECH - 2) // 3, _step, 0)
    for j in (ECH - 2, ECH - 1):
        b = j % 3
        _gather(j, b).wait()
        _scat(j - 1, (j - 1) % 3).wait()
        pltpu.async_copy(rows_v.at[b],
                         m_sh.at[dst_v.at[pl.ds(j * EC, EC)]], ssem[b],
                         add=True)
    _scat(ECH - 1, (ECH - 1) % 3).wait()
    plsc.subcore_barrier()
    pltpu.sync_copy(m_sh.at[pl.ds(s * MRW, MRW)],
                    out_hbm.at[c, pl.ds(s * MRW, MRW)])


@functools.partial(
    pl.kernel,
    out_type=(
        jax.ShapeDtypeStruct((NC, G, H), jnp.float32),
        jax.ShapeDtypeStruct((B, BF), jnp.float32),
    ),
    mesh=_MESH,
    scratch_types=[
        pltpu.VMEM_SHARED((G, H), jnp.float32),
        pltpu.VMEM((PCH, PC), jnp.int32),
        pltpu.VMEM((PC, H), jnp.float32),
        pltpu.VMEM((EW,), jnp.int32),
        pltpu.VMEM((3, EC, BF), jnp.float32),
        pltpu.VMEM((2, EC // 2, BF), jnp.float32),
        pltpu.SemaphoreType.DMA,
        pltpu.SemaphoreType.DMA,
        pltpu.SemaphoreType.DMA,
        pltpu.SemaphoreType.DMA,
        pltpu.SemaphoreType.DMA,
    ],
    compiler_params=pltpu.CompilerParams(use_tc_tiling_on_sc=False),
)
def _sc_heads(h_hbm, hp_hbm, batch_hbm, src_hbm, g_hbm, ep_hbm,
              g_sh, bidx_v, prow_v, sidx_v, ebuf_v, eo_v,
              esem0, esem1, esem2, wsem0, wsem1):
    c = lax.axis_index("c")
    s = lax.axis_index("s")
    w = c * NS + s
    esem = (esem0, esem1, esem2)
    wsem = (wsem0, wsem1)
    zero16 = jnp.zeros((16,), jnp.float32)

    def _zrow(i, carry):
        for j in range(H // 16):
            prow_v[i, pl.ds(j * 16, 16)] = zero16
        return carry

    lax.fori_loop(0, GR, _zrow, 0)
    pltpu.sync_copy(prow_v.at[pl.ds(0, GR)], g_sh.at[pl.ds(s * GR, GR)])
    pltpu.sync_copy(batch_hbm.at[w], bidx_v)
    pltpu.sync_copy(src_hbm.at[w], sidx_v)
    plsc.subcore_barrier()

    def _pstep(k, carry):
        pltpu.sync_copy(h_hbm.at[pl.ds(w * PR + k * PC, PC)], prow_v)
        pltpu.sync_copy(prow_v, g_sh.at[bidx_v.at[k]], add=True)
        return carry

    lax.fori_loop(0, PCH, _pstep, 0)

    def _egather(j, b):
        return pltpu.make_async_copy(
            hp_hbm.at[sidx_v.at[pl.ds(j * EC, EC)]], ebuf_v.at[b], esem[b])

    def _egstart(j, b):
        pltpu.async_copy(
            hp_hbm.at[sidx_v.at[pl.ds(j * EC, EC)]], ebuf_v.at[b], esem[b])

    def _wdesc(j, bo):
        return pltpu.make_async_copy(
            eo_v.at[bo],
            ep_hbm.at[pl.ds(w * (B // NW) + j * (EC // 2), EC // 2)],
            wsem[bo])

    def _esum_out(j, b, bo, first):
        if not first:
            _wdesc(j - 2, bo).wait()
        for r in range(EC // 2):
            eo_v[bo, r] = ebuf_v[b, 2 * r] + ebuf_v[b, 2 * r + 1]
        pltpu.async_copy(
            eo_v.at[bo],
            ep_hbm.at[pl.ds(w * (B // NW) + j * (EC // 2), EC // 2)],
            wsem[bo])

    _egstart(0, 0)
    _egstart(1, 1)
    for j in (0, 1):
        _egather(j, j).wait()
        _egstart(j + 2, (j + 2) % 3)
        _esum_out(j, j, j % 2, True)

    def _estep(jj, carry):
        for b in range(6):
            j = jj * 6 + b + 2
            _egather(j, (b + 2) % 3).wait()
            _egstart(j + 2, (b + 1) % 3)
            _esum_out(j, (b + 2) % 3, b % 2, False)
        return carry

    lax.fori_loop(0, (ECH - 5) // 6, _estep, 0)
    for j in (ECH - 3, ECH - 2, ECH - 1):
        b = j % 3
        _egather(j, b).wait()
        if j + 2 < ECH:
            _egstart(j + 2, (j + 2) % 3)
        _esum_out(j, b, j % 2, False)
    _wdesc(ECH - 2, (ECH - 2) % 2).wait()
    _wdesc(ECH - 1, (ECH - 1) % 2).wait()

    plsc.subcore_barrier()
    pltpu.sync_copy(g_sh.at[pl.ds(s * GR, GR)], g_hbm.at[c, pl.ds(s * GR, GR)])


def kernel(x, edge_index, rev_edge_index, batch,
           W_in, b_in, W_h, b_h, W_node, b_node,
           W_edge, b_edge, W_g1, b_g1, W_g2, b_g2):
    del rev_edge_index
    src2 = edge_index[0].astype(jnp.int32).reshape(NW, EW)
    dst2 = edge_index[1].astype(jnp.int32).reshape(NW, EW)
    batch3 = jnp.concatenate(
        [batch.astype(jnp.int32), jnp.zeros((NP - N,), jnp.int32)]
    ).reshape(NW, PCH, PC)
    xp = jnp.pad(x.astype(jnp.float32), ((0, NP - N), (0, 0)))

    h = _tc_in(xp, W_in, b_in.reshape(1, H))
    for _ in range(DEPTH):
        parts = _sc_msg(h, src2, dst2)
        h = _tc_round(h, parts, parts, W_h, b_h.reshape(1, H))

    node_pred, hp = _tc_heads(
        h, W_node, b_node.reshape(1, AF), W_edge, b_edge.reshape(1, BF))
    g, edge_pred = _sc_heads(h, hp, batch3, src2)
    gfull = _tc_graph(
        g, g, W_g1, b_g1.reshape(1, H),
        jnp.pad(W_g2, ((0, 0), (0, H - 1))),
        jnp.pad(b_g2, (0, H - 1)).reshape(1, H))
    return node_pred[:N], edge_pred, gfull[:, :1]


# speedup vs baseline: 1.4410x; 1.0187x over previous
import functools

import jax
import jax.numpy as jnp
from jax import lax
from jax.experimental import pallas as pl
from jax.experimental.pallas import tpu as pltpu
from jax.experimental.pallas import tpu_sc as plsc

N = 10000
E = 320000
H = 128
AF = 128
BF = 16
G = 256
DEPTH = 3

NC = 2
NS = 16
NW = NC * NS

NP = 10240
BR = 2048
EC = 80
ECH = E // NW // EC
EW = E // NW
NCEIL = N
MRW = NCEIL // NS
B = E // 2
PC = 80
PCH = NP // NW // PC
PR = NP // NW
GR = G // NS

_MESH = plsc.VectorSubcoreMesh(
    core_axis_name="c", subcore_axis_name="s", num_cores=NC, num_subcores=NS)


def _row_mask(val):
    rows = lax.broadcasted_iota(jnp.int32, val.shape, 0) + pl.program_id(0) * BR
    return jnp.where(rows < N, val, 0.0)


def _in_body(x_ref, w_ref, b_ref, o_ref):
    v = jnp.dot(x_ref[...], w_ref[...], preferred_element_type=jnp.float32)
    o_ref[...] = _row_mask(jnp.maximum(v + b_ref[...], 0.0))


_tc_in = pl.pallas_call(
    _in_body,
    grid=(NP // BR,),
    in_specs=[
        pl.BlockSpec((BR, AF), lambda i: (i, 0)),
        pl.BlockSpec((AF, H), lambda i: (0, 0)),
        pl.BlockSpec((1, H), lambda i: (0, 0)),
    ],
    out_specs=pl.BlockSpec((BR, H), lambda i: (i, 0)),
    out_shape=jax.ShapeDtypeStruct((NP, H), jnp.float32),
)


def _round_body(h_ref, m0_ref, m1_ref, w_ref, b_ref, o_ref):
    m = m0_ref[0] + m1_ref[0]
    v = jnp.dot(m, w_ref[...], preferred_element_type=jnp.float32)
    o_ref[...] = _row_mask(jnp.maximum(h_ref[...] + v + b_ref[...], 0.0))


_tc_round = pl.pallas_call(
    _round_body,
    grid=(NP // BR,),
    in_specs=[
        pl.BlockSpec((BR, H), lambda i: (i, 0)),
        pl.BlockSpec((1, BR, H), lambda i: (0, i, 0)),
        pl.BlockSpec((1, BR, H), lambda i: (1, i, 0)),
        pl.BlockSpec((H, H), lambda i: (0, 0)),
        pl.BlockSpec((1, H), lambda i: (0, 0)),
    ],
    out_specs=pl.BlockSpec((BR, H), lambda i: (i, 0)),
    out_shape=jax.ShapeDtypeStruct((NP, H), jnp.float32),
)


def _heads_body(h_ref, wn_ref, bn_ref, we_ref, be_ref, np_ref, hp_ref):
    hh = h_ref[...]
    np_ref[...] = jnp.dot(hh, wn_ref[...], preferred_element_type=jnp.float32) + bn_ref[...]
    ep = jnp.dot(hh, we_ref[...], preferred_element_type=jnp.float32)
    hp_ref[...] = 0.5 * (ep + be_ref[...])


_tc_heads = pl.pallas_call(
    _heads_body,
    grid=(NP // BR,),
    in_specs=[
        pl.BlockSpec((BR, H), lambda i: (i, 0)),
        pl.BlockSpec((H, AF), lambda i: (0, 0)),
        pl.BlockSpec((1, AF), lambda i: (0, 0)),
        pl.BlockSpec((H, BF), lambda i: (0, 0)),
        pl.BlockSpec((1, BF), lambda i: (0, 0)),
    ],
    out_specs=[
        pl.BlockSpec((BR, AF), lambda i: (i, 0)),
        pl.BlockSpec((BR, BF), lambda i: (i, 0)),
    ],
    out_shape=[
        jax.ShapeDtypeStruct((NP, AF), jnp.float32),
        jax.ShapeDtypeStruct((NP, BF), jnp.float32),
    ],
)


def _graph_body(g0_ref, g1_ref, w1_ref, b1_ref, w2_ref, b2_ref, o_ref):
    gsum = g0_ref[0] + g1_ref[0]
    t = jnp.maximum(
        jnp.dot(gsum, w1_ref[...], preferred_element_type=jnp.float32)
        + b1_ref[...], 0.0)
    o_ref[...] = jnp.dot(t, w2_ref[...], preferred_element_type=jnp.float32) + b2_ref[...]


_tc_graph = pl.pallas_call(
    _graph_body,
    grid=(1,),
    in_specs=[
        pl.BlockSpec((1, G, H), lambda i: (0, 0, 0)),
        pl.BlockSpec((1, G, H), lambda i: (1, 0, 0)),
        pl.BlockSpec((H, H), lambda i: (0, 0)),
        pl.BlockSpec((1, H), lambda i: (0, 0)),
        pl.BlockSpec((H, H), lambda i: (0, 0)),
        pl.BlockSpec((1, H), lambda i: (0, 0)),
    ],
    out_specs=pl.BlockSpec((G, H), lambda i: (0, 0)),
    out_shape=jax.ShapeDtypeStruct((G, H), jnp.float32),
)


@functools.partial(
    pl.kernel,
    out_type=jax.ShapeDtypeStruct((NC, NP, H), jnp.float32),
    mesh=_MESH,
    scratch_types=[
        pltpu.VMEM_SHARED((NCEIL, H), jnp.float32),
        pltpu.VMEM((EW,), jnp.int32),
        pltpu.VMEM((EW,), jnp.int32),
        pltpu.VMEM((3, EC, H), jnp.float32),
        pltpu.SemaphoreType.DMA,
        pltpu.SemaphoreType.DMA,
        pltpu.SemaphoreType.DMA,
        pltpu.SemaphoreType.DMA,
        pltpu.SemaphoreType.DMA,
        pltpu.SemaphoreType.DMA,
    ],
    compiler_params=pltpu.CompilerParams(use_tc_tiling_on_sc=False),
)
def _sc_msg(h_hbm, ei_hbm, out_hbm, m_sh, src_v, dst_v, rows_v,
            gsem0, gsem1, gsem2, ssem0, ssem1, ssem2):
    c = lax.axis_index("c")
    s = lax.axis_index("s")
    w = c * NS + s
    gsem = (gsem0, gsem1, gsem2)
    ssem = (ssem0, ssem1, ssem2)

    zero16 = jnp.zeros((16,), jnp.float32)

    def _zrow(i, carry):
        for j in range(H // 16):
            rows_v[0, i, pl.ds(j * 16, 16)] = zero16
        return carry

    lax.fori_loop(0, EC, _zrow, 0)
    for k in range(MRW // EC):
        pltpu.sync_copy(rows_v.at[0], m_sh.at[pl.ds(s * MRW + k * EC, EC)])
    rem = MRW % EC
    if rem:
        pltpu.sync_copy(rows_v.at[0, pl.ds(0, rem), :],
                        m_sh.at[pl.ds(s * MRW + (MRW // EC) * EC, rem)])
    pltpu.sync_copy(ei_hbm.at[0, pl.ds(w * EW, EW)], src_v)
    pltpu.sync_copy(ei_hbm.at[1, pl.ds(w * EW, EW)], dst_v)
    plsc.subcore_barrier()

    def _gather(j, b):
        return pltpu.make_async_copy(
            h_hbm.at[src_v.at[pl.ds(j * EC, EC)]], rows_v.at[b], gsem[b])

    def _gstart(j, b):
        pltpu.async_copy(
            h_hbm.at[src_v.at[pl.ds(j * EC, EC)]], rows_v.at[b], gsem[b])

    def _scat(j, b):
        return pltpu.make_async_copy(
            rows_v.at[b], m_sh.at[dst_v.at[pl.ds(j * EC, EC)]], ssem[b])

    _gstart(0, 0)
    _gstart(1, 1)

    def _step(jj, carry):
        for b in (0, 1, 2):
            j = jj * 3 + b
            _gather(j, b).wait()

            @pl.when(j >= 1)
            def _():
                _scat(j - 1, (b - 1) % 3).wait()

            _gstart(j + 2, (b + 2) % 3)
            pltpu.async_copy(rows_v.at[b],
                             m_sh.at[dst_v.at[pl.ds(j * EC, EC)]], ssem[b],
                             add=True)
        return carry

    lax.fori_loop(0, (ECH - 2) // 3, _step, 0)
    for j in (ECH - 2, ECH - 1):
        b = j % 3
        _gather(j, b).wait()
        _scat(j - 1, (j - 1) % 3).wait()
        pltpu.async_copy(rows_v.at[b],
                         m_sh.at[dst_v.at[pl.ds(j * EC, EC)]], ssem[b],
                         add=True)
    _scat(ECH - 1, (ECH - 1) % 3).wait()
    plsc.subcore_barrier()
    pltpu.sync_copy(m_sh.at[pl.ds(s * MRW, MRW)],
                    out_hbm.at[c, pl.ds(s * MRW, MRW)])


@functools.partial(
    pl.kernel,
    out_type=(
        jax.ShapeDtypeStruct((NC, G, H), jnp.float32),
        jax.ShapeDtypeStruct((B, BF), jnp.float32),
    ),
    mesh=_MESH,
    scratch_types=[
        pltpu.VMEM_SHARED((G, H), jnp.float32),
        pltpu.VMEM((PCH, PC), jnp.int32),
        pltpu.VMEM((PC, H), jnp.float32),
        pltpu.VMEM((EW,), jnp.int32),
        pltpu.VMEM((3, EC, BF), jnp.float32),
        pltpu.VMEM((2, EC // 2, BF), jnp.float32),
        pltpu.SemaphoreType.DMA,
        pltpu.SemaphoreType.DMA,
        pltpu.SemaphoreType.DMA,
        pltpu.SemaphoreType.DMA,
        pltpu.SemaphoreType.DMA,
    ],
    compiler_params=pltpu.CompilerParams(use_tc_tiling_on_sc=False),
)
def _sc_heads(h_hbm, hp_hbm, batch_hbm, ei_hbm, g_hbm, ep_hbm,
              g_sh, bidx_v, prow_v, sidx_v, ebuf_v, eo_v,
              esem0, esem1, esem2, wsem0, wsem1):
    c = lax.axis_index("c")
    s = lax.axis_index("s")
    w = c * NS + s
    esem = (esem0, esem1, esem2)
    wsem = (wsem0, wsem1)
    zero16 = jnp.zeros((16,), jnp.float32)

    def _zrow(i, carry):
        for j in range(H // 16):
            prow_v[i, pl.ds(j * 16, 16)] = zero16
        return carry

    lax.fori_loop(0, GR, _zrow, 0)
    pltpu.sync_copy(prow_v.at[pl.ds(0, GR)], g_sh.at[pl.ds(s * GR, GR)])
    pltpu.sync_copy(batch_hbm.at[w], bidx_v)
    pltpu.sync_copy(ei_hbm.at[0, pl.ds(w * EW, EW)], sidx_v)
    plsc.subcore_barrier()

    def _pstep(k, carry):
        pltpu.sync_copy(h_hbm.at[pl.ds(w * PR + k * PC, PC)], prow_v)
        pltpu.sync_copy(prow_v, g_sh.at[bidx_v.at[k]], add=True)
        return carry

    lax.fori_loop(0, PCH, _pstep, 0)

    def _egather(j, b):
        return pltpu.make_async_copy(
            hp_hbm.at[sidx_v.at[pl.ds(j * EC, EC)]], ebuf_v.at[b], esem[b])

    def _egstart(j, b):
        pltpu.async_copy(
            hp_hbm.at[sidx_v.at[pl.ds(j * EC, EC)]], ebuf_v.at[b], esem[b])

    def _wdesc(j, bo):
        return pltpu.make_async_copy(
            eo_v.at[bo],
            ep_hbm.at[pl.ds(w * (B // NW) + j * (EC // 2), EC // 2)],
            wsem[bo])

    def _esum_out(j, b, bo, first):
        if not first:
            _wdesc(j - 2, bo).wait()
        for r in range(EC // 2):
            eo_v[bo, r] = ebuf_v[b, 2 * r] + ebuf_v[b, 2 * r + 1]
        pltpu.async_copy(
            eo_v.at[bo],
            ep_hbm.at[pl.ds(w * (B // NW) + j * (EC // 2), EC // 2)],
            wsem[bo])

    _egstart(0, 0)
    _egstart(1, 1)
    for j in (0, 1):
        _egather(j, j).wait()
        _egstart(j + 2, (j + 2) % 3)
        _esum_out(j, j, j % 2, True)

    def _estep(jj, carry):
        for b in range(6):
            j = jj * 6 + b + 2
            _egather(j, (b + 2) % 3).wait()
            _egstart(j + 2, (b + 1) % 3)
            _esum_out(j, (b + 2) % 3, b % 2, False)
        return carry

    lax.fori_loop(0, (ECH - 5) // 6, _estep, 0)
    for j in (ECH - 3, ECH - 2, ECH - 1):
        b = j % 3
        _egather(j, b).wait()
        if j + 2 < ECH:
            _egstart(j + 2, (j + 2) % 3)
        _esum_out(j, b, j % 2, False)
    _wdesc(ECH - 2, (ECH - 2) % 2).wait()
    _wdesc(ECH - 1, (ECH - 1) % 2).wait()

    plsc.subcore_barrier()
    pltpu.sync_copy(g_sh.at[pl.ds(s * GR, GR)], g_hbm.at[c, pl.ds(s * GR, GR)])


def kernel(x, edge_index, rev_edge_index, batch,
           W_in, b_in, W_h, b_h, W_node, b_node,
           W_edge, b_edge, W_g1, b_g1, W_g2, b_g2):
    del rev_edge_index
    ei = edge_index.astype(jnp.int32)
    batch3 = jnp.concatenate(
        [batch.astype(jnp.int32), jnp.zeros((NP - N,), jnp.int32)]
    ).reshape(NW, PCH, PC)
    xp = jnp.pad(x.astype(jnp.float32), ((0, NP - N), (0, 0)))

    h = _tc_in(xp, W_in, b_in.reshape(1, H))
    for _ in range(DEPTH):
        parts = _sc_msg(h, ei)
        h = _tc_round(h, parts, parts, W_h, b_h.reshape(1, H))

    node_pred, hp = _tc_heads(
        h, W_node, b_node.reshape(1, AF), W_edge, b_edge.reshape(1, BF))
    g, edge_pred = _sc_heads(h, hp, batch3, ei)
    gfull = _tc_graph(
        g, g, W_g1, b_g1.reshape(1, H),
        jnp.pad(W_g2, ((0, 0), (0, H - 1))),
        jnp.pad(b_g2, (0, H - 1)).reshape(1, H))
    return node_pred[:N], edge_pred, gfull[:, :1]
